# baseline (speedup 1.0000x reference)
import functools

import jax
import jax.numpy as jnp
from jax import lax
from jax.experimental import pallas as pl
from jax.experimental.pallas import tpu as pltpu

E = 320000
DIM = 128
V0, V1, V2 = 5, 6, 2
NT = V0 * V1 * V2
TPAD = 64
B = 1000
NB = E // B


def _tc_body(a0_ref, a1_ref, a2_ref, e0_ref, e1_ref, e2_ref, msg_ref, out_ref):
    c = (V1 * V2) * a0_ref[0, 0, :] + V2 * a1_ref[0, 0, :] + a2_ref[0, 0, :]
    onehot = (c[:, None] == lax.broadcasted_iota(jnp.int32, (1, TPAD), 1)
              ).astype(jnp.float32)
    e0 = e0_ref[...]
    e1 = e1_ref[...]
    e2 = e2_ref[...]
    t = (e0[:, None, None, :] + e1[None, :, None, :]
         + e2[None, None, :, :]).reshape(NT, DIM)
    t64 = jnp.concatenate([t, jnp.zeros((TPAD - NT, DIM), jnp.float32)], axis=0)
    bond = jnp.dot(onehot, t64, preferred_element_type=jnp.float32)
    out_ref[...] = jnp.maximum(msg_ref[...] + bond, 0.0)


def kernel(message, attrs, emb0, emb1, emb2):
    a = attrs.astype(jnp.int32)
    a0 = a[:, 0].reshape(NB, 1, B)
    a1 = a[:, 1].reshape(NB, 1, B)
    a2 = a[:, 2].reshape(NB, 1, B)
    full = lambda shape: pl.BlockSpec(shape, lambda i: (0,) * len(shape))
    return pl.pallas_call(
        _tc_body,
        grid=(NB,),
        in_specs=[
            pl.BlockSpec((1, 1, B), lambda i: (i, 0, 0)),
            pl.BlockSpec((1, 1, B), lambda i: (i, 0, 0)),
            pl.BlockSpec((1, 1, B), lambda i: (i, 0, 0)),
            full((V0, DIM)),
            full((V1, DIM)),
            full((V2, DIM)),
            pl.BlockSpec((B, DIM), lambda i: (i, 0)),
        ],
        out_specs=pl.BlockSpec((B, DIM), lambda i: (i, 0)),
        out_shape=jax.ShapeDtypeStruct((E, DIM), jnp.float32),
    )(a0, a1, a2, emb0, emb1, emb2, message)
